# baseline (speedup 1.0000x reference)
import jax
import jax.numpy as jnp
from jax.experimental import pallas as pl


def _make_body(BS):
    def _add_body(x_ref, pos_ref, out_ref):
        i = pl.program_id(0)
        pos = pos_ref[pl.ds(i * BS, BS), :]
        out_ref[...] = x_ref[...] + pos[:, None, :]
    return _add_body


def kernel(x, pos_table):
    S, B, E = x.shape
    BS = 4
    grid = (S // BS,)
    return pl.pallas_call(
        _make_body(BS),
        grid=grid,
        in_specs=[
            pl.BlockSpec((BS, B, E), lambda i: (i, 0, 0)),
            pl.BlockSpec((S, E), lambda i: (0, 0)),
        ],
        out_specs=pl.BlockSpec((BS, B, E), lambda i: (i, 0, 0)),
        out_shape=jax.ShapeDtypeStruct((S, B, E), x.dtype),
    )(x, pos_table)


# speedup vs baseline: 1.0892x; 1.0892x over previous
import jax
import jax.numpy as jnp
from jax.experimental import pallas as pl


def _make_body(BS):
    def _add_body(x_ref, pos_ref, out_ref):
        i = pl.program_id(0)
        pos = pos_ref[pl.ds(i * BS, BS), :]
        out_ref[...] = x_ref[...] + pos[:, None, :]
    return _add_body


def kernel(x, pos_table):
    S, B, E = x.shape
    BS = 8
    grid = (S // BS,)
    return pl.pallas_call(
        _make_body(BS),
        grid=grid,
        in_specs=[
            pl.BlockSpec((BS, B, E), lambda i: (i, 0, 0)),
            pl.BlockSpec((S, E), lambda i: (0, 0)),
        ],
        out_specs=pl.BlockSpec((BS, B, E), lambda i: (i, 0, 0)),
        out_shape=jax.ShapeDtypeStruct((S, B, E), x.dtype),
    )(x, pos_table)


# speedup vs baseline: 1.0913x; 1.0019x over previous
import jax
import jax.numpy as jnp
from jax.experimental import pallas as pl


def _add_body(x_ref, pos_ref, out_ref):
    out_ref[...] = x_ref[...] + pos_ref[...][:, None, :]


def kernel(x, pos_table):
    S, B, E = x.shape
    BS = 8
    grid = (S // BS,)
    return pl.pallas_call(
        _add_body,
        grid=grid,
        in_specs=[
            pl.BlockSpec((BS, B, E), lambda i: (i, 0, 0)),
            pl.BlockSpec((BS, E), lambda i: (i, 0)),
        ],
        out_specs=pl.BlockSpec((BS, B, E), lambda i: (i, 0, 0)),
        out_shape=jax.ShapeDtypeStruct((S, B, E), x.dtype),
    )(x, pos_table)
